# baseline (speedup 1.0000x reference)
import functools

import numpy as np
import jax
import jax.numpy as jnp
from jax import lax
from jax.experimental import pallas as pl
from jax.experimental.pallas import tpu as pltpu
from jax.experimental.pallas import tpu_sc as plsc

_MAX_LEN = 200
_EMB_DIM = 64
_NW = 32
_NBUF = 4
_LAG = 2
_BB = 128
_TSLAB = 8


def _make_pe_np():
    pos = np.expand_dims(np.arange(_MAX_LEN), 1)
    pe = pos / np.power(
        1000, 2 * np.expand_dims(np.arange(_EMB_DIM) // 2, 0) / _EMB_DIM
    )
    pe = pe.astype(np.float64)
    pe[:, 0::2] = np.sin(pe[:, 0::2])
    pe[:, 1::2] = np.cos(pe[:, 1::2])
    return pe.astype(np.float32)


_PE = _make_pe_np()


def _emb_sc(table2, xt, pe):
    n_b = xt.shape[1]
    bpw = n_b // _NW
    nbb = bpw // _BB
    n_chunks = _MAX_LEN * nbb
    n_rounds = _MAX_LEN
    n_slabs = _MAX_LEN // _TSLAB
    mesh = plsc.VectorSubcoreMesh(core_axis_name="c", subcore_axis_name="s")

    @functools.partial(
        pl.kernel,
        mesh=mesh,
        out_type=jax.ShapeDtypeStruct((_MAX_LEN, _EMB_DIM, n_b), jnp.float32),
        scratch_types=[
            pltpu.VMEM((2, _TSLAB, bpw), jnp.int32),
            pltpu.VMEM((2, _TSLAB, bpw), jnp.int32),
            pltpu.VMEM((_MAX_LEN, _EMB_DIM), jnp.float32),
            pltpu.VMEM((_NBUF, _BB, 2 * _EMB_DIM), jnp.float32),
            pltpu.VMEM((2, _EMB_DIM, _BB), jnp.float32),
            pltpu.SemaphoreType.DMA,
            pltpu.SemaphoreType.DMA((_NBUF,)),
            pltpu.SemaphoreType.DMA((2,)),
        ],
        compiler_params=pltpu.CompilerParams(needs_layout_passes=False),
    )
    def k(tab_h, xt_h, pe_h, out_h, idx_v, idx2_v, pe_v, rows_v, slab_v,
          sem_ix, sem_g, sem_o):
        cid = lax.axis_index("c")
        sid = lax.axis_index("s")
        wid = sid * 2 + cid
        bw0 = wid * bpw
        pltpu.sync_copy(pe_h, pe_v)

        row16 = lax.iota(jnp.int32, 16)
        rowc = [row16 + 16 * j for j in range(4)]

        def stage_idx(s_blk, buf, sync):
            src = xt_h.at[pl.ds(s_blk * _TSLAB, _TSLAB), pl.ds(bw0, bpw)]
            if sync:
                pltpu.sync_copy(src, idx_v.at[buf])
            else:
                pltpu.async_copy(src, idx_v.at[buf], sem_ix)

        def wait_idx(buf):
            pltpu.make_async_copy(
                xt_h.at[pl.ds(0, _TSLAB), pl.ds(bw0, bpw)], idx_v.at[buf], sem_ix
            ).wait()

        def shift_idx(buf):
            def vec_it(i, carry):
                for tt in range(_TSLAB):
                    sl = pl.ds(i * 16, 16)
                    idx2_v[buf, tt, sl] = lax.shift_right_logical(
                        idx_v[buf, tt, sl], 1
                    )
                return carry

            lax.fori_loop(0, bpw // 16, vec_it, 0)

        def start_gather(g, buf, tt, bb):
            pltpu.async_copy(
                tab_h.at[idx2_v.at[buf, tt, pl.ds(bb * _BB, _BB)]],
                rows_v.at[g],
                sem_g.at[g],
            )

        def wait_gather(g):
            pltpu.make_async_copy(
                tab_h.at[idx2_v.at[0, 0, pl.ds(0, _BB)]],
                rows_v.at[g],
                sem_g.at[g],
            ).wait()

        def start_store(ss, t_cd, bb_cd):
            pltpu.async_copy(
                slab_v.at[ss],
                out_h.at[t_cd, :, pl.ds(bw0 + bb_cd * _BB, _BB)],
                sem_o.at[ss],
            )

        def wait_store(ss):
            pltpu.make_async_copy(
                slab_v.at[ss], out_h.at[0, :, pl.ds(0, _BB)], sem_o.at[ss]
            ).wait()

        def transpose_pe(g, ss, t_cd, buf_cd, tt_cd, bb_cd):
            pv = [pe_v[t_cd, pl.ds(16 * j, 16)] for j in range(4)]

            def b_it(i, carry):
                rawv = idx_v[buf_cd, tt_cd, pl.ds(bb_cd * _BB + i * 16, 16)]
                hv = lax.rem(rawv, 2) * _EMB_DIM
                for u in range(16):
                    bi = i * 16 + u
                    half = hv[u]
                    col = jnp.broadcast_to(bi, (16,)).astype(jnp.int32)
                    for j in range(4):
                        v = rows_v[g, bi, pl.ds(half + 16 * j, 16)] + pv[j]
                        plsc.store_scatter(slab_v.at[ss], [rowc[j], col], v)
                return carry

            lax.fori_loop(0, _BB // 16, b_it, 0)

        stage_idx(0, 0, True)
        stage_idx(1, 1, False)
        shift_idx(0)

        for b in range(_NBUF):
            start_gather(b, 0, 0, b)
            if b >= _LAG:
                cd = b - _LAG
                wait_gather(cd % _NBUF)
                transpose_pe(cd % _NBUF, cd % 2, 0, 0, 0, cd)
                start_store(cd % 2, 0, cd)

        def round_body(r, carry):
            s_blk = r // _TSLAB
            tt = lax.rem(r, _TSLAB)
            buf = lax.rem(s_blk, 2)

            @pl.when(lax.rem(r, _TSLAB) == 0)
            def _():
                wait_idx(buf)
                shift_idx(buf)

            for b in range(_NBUF):
                c = r * _NBUF + b
                start_gather(b, buf, tt, b)
                sg = (b + _LAG) % _NBUF
                ss = b % 2
                bb_cd = (b + _LAG) % _NBUF
                if b < _LAG:
                    t_cd = r - 1
                    tt_cd = lax.rem(t_cd, _TSLAB)
                    buf_cd = lax.rem(t_cd // _TSLAB, 2)
                else:
                    t_cd = r
                    tt_cd = tt
                    buf_cd = buf
                wait_gather(sg)
                wait_store(ss)
                transpose_pe(sg, ss, t_cd, buf_cd, tt_cd, bb_cd)
                start_store(ss, t_cd, bb_cd)
                if b == _LAG - 1:
                    @pl.when((lax.rem(r, _TSLAB) == 0) & (s_blk < n_slabs - 1))
                    def _():
                        stage_idx(s_blk + 1, 1 - buf, False)
            return carry

        lax.fori_loop(1, n_rounds, round_body, 0)

        last_buf = (n_slabs - 1) % 2
        for e in range(_LAG):
            cd = n_chunks - _LAG + e
            sg = cd % _NBUF
            ss = cd % 2
            wait_gather(sg)
            wait_store(ss)
            transpose_pe(
                sg, ss, _MAX_LEN - 1, last_buf, _TSLAB - 1, cd % _NBUF
            )
            start_store(ss, _MAX_LEN - 1, cd % _NBUF)
        for ss in range(2):
            wait_store(ss)

    return k(table2, xt, pe)


def kernel(x, table):
    xt = jnp.transpose(x.astype(jnp.int32))
    table2 = table.reshape(table.shape[0] // 2, 2 * _EMB_DIM)
    pe = jnp.asarray(_PE)
    out_t = _emb_sc(table2, xt, pe)
    return jnp.transpose(out_t, (2, 0, 1))
